# baseline (speedup 1.0000x reference)
import functools
import math

import jax
import jax.numpy as jnp
from jax.experimental import pallas as pl
from jax.experimental.pallas import tpu as pltpu

S = 2048
H = 1024
NH = 16
DH = H // NH
E = 64
DFF = 1024
CAP = 40
EC = E * CAP
SB = 256


def _ln(x, w, b, eps=1e-5):
    mu = jnp.mean(x, axis=-1, keepdims=True)
    var = jnp.mean((x - mu) ** 2, axis=-1, keepdims=True)
    return (x - mu) * jax.lax.rsqrt(var + eps) * w + b


def _qkv_body(x_ref, w_ref, lw_ref, lb_ref, o_ref):
    x = _ln(x_ref[...], lw_ref[...], lb_ref[...]).astype(jnp.bfloat16)
    o_ref[...] = jax.lax.dot_general(
        x, w_ref[...], (((1,), (1,)), ((), ())),
        preferred_element_type=jnp.float32).astype(jnp.bfloat16)


def _qkv(hid, qkv_w, lw, lb):
    return pl.pallas_call(
        _qkv_body,
        grid=(S // SB,),
        in_specs=[
            pl.BlockSpec((SB, H), lambda i: (i, 0)),
            pl.BlockSpec((3 * H, H), lambda i: (0, 0)),
            pl.BlockSpec((1, H), lambda i: (0, 0)),
            pl.BlockSpec((1, H), lambda i: (0, 0)),
        ],
        out_specs=pl.BlockSpec((SB, 3 * H), lambda i: (i, 0)),
        out_shape=jax.ShapeDtypeStruct((S, 3 * H), jnp.bfloat16),
    )(hid, qkv_w, lw, lb)


def _attn_body(q_ref, k_ref, v_ref, o_ref):
    i = pl.program_id(1)
    scale = 1.0 / math.sqrt(DH)
    row = jax.lax.broadcasted_iota(jnp.int32, (SB, SB), 0)
    col = jax.lax.broadcasted_iota(jnp.int32, (SB, SB), 1)
    causal = col <= row
    for sub in range(2):
        sl = slice(sub * DH, (sub + 1) * DH)
        q = q_ref[:, sl]
        kd = k_ref[pl.ds(i * SB, SB), sl]
        vd = v_ref[pl.ds(i * SB, SB), sl]
        s = jax.lax.dot_general(q, kd, (((1,), (1,)), ((), ())),
                                preferred_element_type=jnp.float32) * scale
        s = jnp.where(causal, s, jnp.float32(-1e9))
        m = jnp.max(s, axis=-1, keepdims=True)
        p = jnp.exp(s - m)
        l = jnp.sum(p, axis=-1, keepdims=True)
        acc = jax.lax.dot_general(p.astype(jnp.bfloat16), vd,
                                  (((1,), (0,)), ((), ())),
                                  preferred_element_type=jnp.float32)

        def body(j, carry):
            m, l, acc = carry
            kt = k_ref[pl.ds(j * SB, SB), sl]
            vt = v_ref[pl.ds(j * SB, SB), sl]
            s = jax.lax.dot_general(q, kt, (((1,), (1,)), ((), ())),
                                    preferred_element_type=jnp.float32) * scale
            mt = jnp.max(s, axis=-1, keepdims=True)
            mn = jnp.maximum(m, mt)
            p = jnp.exp(s - mn)
            corr = jnp.exp(m - mn)
            ln = l * corr + jnp.sum(p, axis=-1, keepdims=True)
            accn = acc * corr + jax.lax.dot_general(
                p.astype(jnp.bfloat16), vt, (((1,), (0,)), ((), ())),
                preferred_element_type=jnp.float32)
            return mn, ln, accn

        m, l, acc = jax.lax.fori_loop(0, i, body, (m, l, acc))
        o_ref[:, sl] = acc / l


def _attn(qkv):
    hp = NH // 2
    return pl.pallas_call(
        _attn_body,
        grid=(hp, S // SB),
        in_specs=[
            pl.BlockSpec((SB, 2 * DH), lambda h, i: (i, h)),
            pl.BlockSpec((S, 2 * DH), lambda h, i: (0, hp + h)),
            pl.BlockSpec((S, 2 * DH), lambda h, i: (0, 2 * hp + h)),
        ],
        out_specs=pl.BlockSpec((SB, 2 * DH), lambda h, i: (i, h)),
        out_shape=jax.ShapeDtypeStruct((S, H), jnp.float32),
    )(qkv, qkv, qkv)


def _post_body(hid_ref, ao_ref, pw_ref, rw_ref, lw_ref, lb_ref,
               hattn_ref, ln2_ref, dslot_ref, pscale_ref):
    proj = jax.lax.dot_general(ao_ref[...], pw_ref[...],
                               (((1,), (1,)), ((), ())),
                               preferred_element_type=jnp.float32)
    h_attn = hid_ref[...] + proj
    hattn_ref[...] = h_attn
    ln2 = _ln(h_attn, lw_ref[...], lb_ref[...])
    ln2_ref[...] = ln2
    logits = jax.lax.dot_general(ln2, rw_ref[...], (((1,), (1,)), ((), ())),
                                 preferred_element_type=jnp.float32)
    lmax = jnp.max(logits, axis=-1, keepdims=True)
    p = 1.0 / jnp.sum(jnp.exp(logits - lmax), axis=-1, keepdims=True)
    eiota = jax.lax.broadcasted_iota(jnp.int32, (S, E), 1)
    eidx = jnp.min(jnp.where(logits == lmax, eiota, E), axis=-1,
                   keepdims=True)
    oh = (eiota == eidx).astype(jnp.bfloat16)
    rown = jax.lax.broadcasted_iota(jnp.int32, (S, S), 0)
    coln = jax.lax.broadcasted_iota(jnp.int32, (S, S), 1)
    tril = (coln < rown).astype(jnp.bfloat16)
    cnt = jax.lax.dot_general(tril, oh, (((1,), (0,)), ((), ())),
                              preferred_element_type=jnp.float32)
    oh = oh.astype(jnp.float32)
    pos = jnp.sum(cnt * oh, axis=-1, keepdims=True).astype(jnp.int32)
    keep = pos < CAP
    slot = eidx * CAP + pos
    dslot_ref[...] = jnp.where(keep, slot, -1)
    pscale_ref[...] = jnp.where(keep, p, 0.0)


def _post(hid, attn_out, proj_w, router_w, lw, lb):
    return pl.pallas_call(
        _post_body,
        out_shape=(
            jax.ShapeDtypeStruct((S, H), jnp.float32),
            jax.ShapeDtypeStruct((S, H), jnp.float32),
            jax.ShapeDtypeStruct((S, 1), jnp.int32),
            jax.ShapeDtypeStruct((S, 1), jnp.float32),
        ),
    )(hid, attn_out, proj_w, router_w, lw, lb)


def _disp_body(dslotT_ref, ln2_ref, xe_ref):
    siota = jax.lax.broadcasted_iota(jnp.int32, (EC, S), 0)
    dt = (dslotT_ref[...] == siota).astype(jnp.float32)
    xe_ref[...] = jax.lax.dot_general(dt, ln2_ref[...],
                                      (((1,), (0,)), ((), ())),
                                      preferred_element_type=jnp.float32)


def _dispatch(dslotT, ln2):
    return pl.pallas_call(
        _disp_body,
        out_shape=jax.ShapeDtypeStruct((EC, H), jnp.float32),
    )(dslotT, ln2)


def _ffn_body(xe_ref, w1_ref, w2_ref, ye_ref):
    x = xe_ref[...]
    h = jax.lax.dot_general(x, w1_ref[0], (((1,), (1,)), ((), ())),
                            preferred_element_type=jnp.float32)
    inner = 0.7978845608028654 * (h + 0.044715 * (h * h * h))
    g = 0.5 * h * (1.0 + jnp.tanh(inner))
    ye_ref[...] = jax.lax.dot_general(g, w2_ref[0], (((1,), (1,)), ((), ())),
                                      preferred_element_type=jnp.float32)


def _ffn(xe, w1, w2):
    return pl.pallas_call(
        _ffn_body,
        grid=(E,),
        in_specs=[
            pl.BlockSpec((CAP, H), lambda e: (e, 0)),
            pl.BlockSpec((1, DFF, H), lambda e: (e, 0, 0)),
            pl.BlockSpec((1, H, DFF), lambda e: (e, 0, 0)),
        ],
        out_specs=pl.BlockSpec((CAP, H), lambda e: (e, 0)),
        out_shape=jax.ShapeDtypeStruct((EC, H), jnp.float32),
    )(xe, w1, w2)


def _comb_body(dslot_ref, pscale_ref, ye_ref, hattn_ref, o_ref):
    siota = jax.lax.broadcasted_iota(jnp.int32, (S, EC), 1)
    dp = (dslot_ref[...] == siota).astype(jnp.float32) * pscale_ref[...]
    comb = jax.lax.dot_general(dp, ye_ref[...], (((1,), (0,)), ((), ())),
                               preferred_element_type=jnp.float32)
    o_ref[...] = hattn_ref[...] + comb


def _combine(dslot, pscale, ye, h_attn):
    return pl.pallas_call(
        _comb_body,
        out_shape=jax.ShapeDtypeStruct((S, H), jnp.float32),
    )(dslot, pscale, ye, h_attn)


def kernel(hidden_states, ln1_weight, ln1_bias, ln2_weight, ln2_bias,
           qkv_weight, proj_weight, router_weight, moe_w1, moe_w2):
    hid = hidden_states.reshape(S, H)
    qkv = _qkv(hid, qkv_weight.astype(jnp.bfloat16),
               ln1_weight.reshape(1, H), ln1_bias.reshape(1, H))
    attn_out = _attn(qkv)
    h_attn, ln2, dslot, pscale = _post(
        hid, attn_out, proj_weight, router_weight,
        ln2_weight.reshape(1, H), ln2_bias.reshape(1, H))
    xe = _dispatch(dslot.reshape(1, S), ln2)
    ye = _ffn(xe, moe_w1, moe_w2)
    out = _combine(dslot, pscale, ye, h_attn)
    return out.reshape(S, 1, H)


# speedup vs baseline: 1.3230x; 1.3230x over previous
import functools
import math

import jax
import jax.numpy as jnp
from jax.experimental import pallas as pl
from jax.experimental.pallas import tpu as pltpu

S = 2048
H = 1024
NH = 16
DH = H // NH
E = 64
DFF = 1024
CAP = 40
EC = E * CAP
SB = 256


def _ln(x, w, b, eps=1e-5):
    mu = jnp.mean(x, axis=-1, keepdims=True)
    var = jnp.mean((x - mu) ** 2, axis=-1, keepdims=True)
    return (x - mu) * jax.lax.rsqrt(var + eps) * w + b


def _qkv_body(x_ref, w_ref, lw_ref, lb_ref, o_ref):
    x = _ln(x_ref[...], lw_ref[...], lb_ref[...]).astype(jnp.bfloat16)
    o_ref[...] = jax.lax.dot_general(
        x, w_ref[...], (((1,), (1,)), ((), ())),
        preferred_element_type=jnp.float32).astype(jnp.bfloat16)


def _qkv(hid, qkv_w, lw, lb):
    return pl.pallas_call(
        _qkv_body,
        grid=(S // SB,),
        in_specs=[
            pl.BlockSpec((SB, H), lambda i: (i, 0)),
            pl.BlockSpec((3 * H, H), lambda i: (0, 0)),
            pl.BlockSpec((1, H), lambda i: (0, 0)),
            pl.BlockSpec((1, H), lambda i: (0, 0)),
        ],
        out_specs=pl.BlockSpec((SB, 3 * H), lambda i: (i, 0)),
        out_shape=jax.ShapeDtypeStruct((S, 3 * H), jnp.bfloat16),
    )(hid, qkv_w, lw, lb)


def _attn_body(q_ref, k_ref, v_ref, o_ref):
    i = pl.program_id(1)
    scale = 1.0 / math.sqrt(DH)
    row = jax.lax.broadcasted_iota(jnp.int32, (SB, S), 0) + i * SB
    col = jax.lax.broadcasted_iota(jnp.int32, (SB, S), 1)
    causal = col <= row
    for sub in range(2):
        sl = slice(sub * DH, (sub + 1) * DH)
        q = q_ref[:, sl]
        k = k_ref[:, sl]
        v = v_ref[:, sl]
        s = jax.lax.dot_general(q, k, (((1,), (1,)), ((), ())),
                                preferred_element_type=jnp.float32) * scale
        s = jnp.where(causal, s, jnp.float32(-1e9))
        m = jnp.max(s, axis=-1, keepdims=True)
        e = jnp.exp(s - m)
        a = e / jnp.sum(e, axis=-1, keepdims=True)
        o_ref[:, sl] = jax.lax.dot_general(a.astype(jnp.bfloat16), v,
                                           (((1,), (0,)), ((), ())),
                                           preferred_element_type=jnp.float32)


def _attn(qkv):
    hp = NH // 2
    return pl.pallas_call(
        _attn_body,
        grid=(hp, S // SB),
        in_specs=[
            pl.BlockSpec((SB, 2 * DH), lambda h, i: (i, h)),
            pl.BlockSpec((S, 2 * DH), lambda h, i: (0, hp + h)),
            pl.BlockSpec((S, 2 * DH), lambda h, i: (0, 2 * hp + h)),
        ],
        out_specs=pl.BlockSpec((SB, 2 * DH), lambda h, i: (i, h)),
        out_shape=jax.ShapeDtypeStruct((S, H), jnp.float32),
    )(qkv, qkv, qkv)


def _post_body(hid_ref, ao_ref, pw_ref, rw_ref, lw_ref, lb_ref,
               hattn_ref, ln2_ref, dslot_ref, pscale_ref):
    proj = jax.lax.dot_general(ao_ref[...], pw_ref[...],
                               (((1,), (1,)), ((), ())),
                               preferred_element_type=jnp.float32)
    h_attn = hid_ref[...] + proj
    hattn_ref[...] = h_attn
    ln2 = _ln(h_attn, lw_ref[...], lb_ref[...])
    ln2_ref[...] = ln2
    logits = jax.lax.dot_general(ln2, rw_ref[...], (((1,), (1,)), ((), ())),
                                 preferred_element_type=jnp.float32)
    lmax = jnp.max(logits, axis=-1, keepdims=True)
    p = 1.0 / jnp.sum(jnp.exp(logits - lmax), axis=-1, keepdims=True)
    eiota = jax.lax.broadcasted_iota(jnp.int32, (S, E), 1)
    eidx = jnp.min(jnp.where(logits == lmax, eiota, E), axis=-1,
                   keepdims=True)
    oh = (eiota == eidx).astype(jnp.bfloat16)
    rown = jax.lax.broadcasted_iota(jnp.int32, (S, S), 0)
    coln = jax.lax.broadcasted_iota(jnp.int32, (S, S), 1)
    tril = (coln < rown).astype(jnp.bfloat16)
    cnt = jax.lax.dot_general(tril, oh, (((1,), (0,)), ((), ())),
                              preferred_element_type=jnp.float32)
    oh = oh.astype(jnp.float32)
    pos = jnp.sum(cnt * oh, axis=-1, keepdims=True).astype(jnp.int32)
    keep = pos < CAP
    slot = eidx * CAP + pos
    dslot_ref[...] = jnp.where(keep, slot, -1)
    pscale_ref[...] = jnp.where(keep, p, 0.0)


def _post(hid, attn_out, proj_w, router_w, lw, lb):
    return pl.pallas_call(
        _post_body,
        out_shape=(
            jax.ShapeDtypeStruct((S, H), jnp.float32),
            jax.ShapeDtypeStruct((S, H), jnp.float32),
            jax.ShapeDtypeStruct((S, 1), jnp.int32),
            jax.ShapeDtypeStruct((S, 1), jnp.float32),
        ),
    )(hid, attn_out, proj_w, router_w, lw, lb)


def _disp_body(dslotT_ref, ln2_ref, xe_ref):
    siota = jax.lax.broadcasted_iota(jnp.int32, (EC, S), 0)
    dt = (dslotT_ref[...] == siota).astype(jnp.float32)
    xe_ref[...] = jax.lax.dot_general(dt, ln2_ref[...],
                                      (((1,), (0,)), ((), ())),
                                      preferred_element_type=jnp.float32)


def _dispatch(dslotT, ln2):
    return pl.pallas_call(
        _disp_body,
        out_shape=jax.ShapeDtypeStruct((EC, H), jnp.float32),
    )(dslotT, ln2)


def _ffn_body(xe_ref, w1_ref, w2_ref, ye_ref):
    x = xe_ref[...]
    h = jax.lax.dot_general(x, w1_ref[0], (((1,), (1,)), ((), ())),
                            preferred_element_type=jnp.float32)
    inner = 0.7978845608028654 * (h + 0.044715 * (h * h * h))
    g = 0.5 * h * (1.0 + jnp.tanh(inner))
    ye_ref[...] = jax.lax.dot_general(g, w2_ref[0], (((1,), (1,)), ((), ())),
                                      preferred_element_type=jnp.float32)


def _ffn(xe, w1, w2):
    return pl.pallas_call(
        _ffn_body,
        grid=(E,),
        in_specs=[
            pl.BlockSpec((CAP, H), lambda e: (e, 0)),
            pl.BlockSpec((1, DFF, H), lambda e: (e, 0, 0)),
            pl.BlockSpec((1, H, DFF), lambda e: (e, 0, 0)),
        ],
        out_specs=pl.BlockSpec((CAP, H), lambda e: (e, 0)),
        out_shape=jax.ShapeDtypeStruct((EC, H), jnp.float32),
    )(xe, w1, w2)


def _comb_body(dslot_ref, pscale_ref, ye_ref, hattn_ref, o_ref):
    siota = jax.lax.broadcasted_iota(jnp.int32, (S, EC), 1)
    dp = (dslot_ref[...] == siota).astype(jnp.float32) * pscale_ref[...]
    comb = jax.lax.dot_general(dp, ye_ref[...], (((1,), (0,)), ((), ())),
                               preferred_element_type=jnp.float32)
    o_ref[...] = hattn_ref[...] + comb


def _combine(dslot, pscale, ye, h_attn):
    return pl.pallas_call(
        _comb_body,
        out_shape=jax.ShapeDtypeStruct((S, H), jnp.float32),
    )(dslot, pscale, ye, h_attn)


def kernel(hidden_states, ln1_weight, ln1_bias, ln2_weight, ln2_bias,
           qkv_weight, proj_weight, router_weight, moe_w1, moe_w2):
    hid = hidden_states.reshape(S, H)
    qkv = _qkv(hid, qkv_weight.astype(jnp.bfloat16),
               ln1_weight.reshape(1, H), ln1_bias.reshape(1, H))
    attn_out = _attn(qkv)
    h_attn, ln2, dslot, pscale = _post(
        hid, attn_out, proj_weight, router_weight,
        ln2_weight.reshape(1, H), ln2_bias.reshape(1, H))
    xe = _dispatch(dslot.reshape(1, S), ln2)
    ye = _ffn(xe, moe_w1, moe_w2)
    out = _combine(dslot, pscale, ye, h_attn)
    return out.reshape(S, 1, H)
